# baseline (speedup 1.0000x reference)
import functools

import jax
import jax.numpy as jnp
from jax import lax
from jax.experimental import pallas as pl
from jax.experimental.pallas import tpu as pltpu
from jax.experimental.pallas import tpu_sc as plsc

N_TREES = 128
MAX_DEPTH = 10
N_NODES = 2 ** MAX_DEPTH - 1
NPAD = N_NODES + 1
BATCH = 16384
N_FEAT = 512
N_WORDS = N_FEAT // 32

NC = 2
NS = 16
NW = NC * NS
TP = 4
RP = NW // TP
TREES_PER = N_TREES // TP
ROWS_PER = BATCH // RP
CHUNK = 1024
N_CHUNKS = ROWS_PER // CHUNK


def _make_forest_kernel():
  mesh = plsc.VectorSubcoreMesh(
      core_axis_name="c", subcore_axis_name="s", num_cores=NC,
      num_subcores=NS)

  @functools.partial(
      pl.kernel,
      out_type=jax.ShapeDtypeStruct((BATCH, N_TREES), jnp.int32),
      mesh=mesh,
      scratch_types=[
          pltpu.VMEM((TREES_PER, NPAD), jnp.int32),
          pltpu.VMEM((CHUNK * N_WORDS,), jnp.int32),
          pltpu.VMEM((CHUNK, TREES_PER), jnp.int32),
      ],
      compiler_params=pltpu.CompilerParams(
          use_tc_tiling_on_sc=False, needs_layout_passes=False),
  )
  def forest(xw_hbm, tbl_hbm, out_hbm, tbl_v, xw_v, out_v):
    wid = lax.axis_index("s") * NC + lax.axis_index("c")
    tp = wid // RP
    rp = wid % RP

    pltpu.sync_copy(tbl_hbm.at[pl.ds(tp * TREES_PER, TREES_PER), :], tbl_v)

    lane = lax.iota(jnp.int32, 16)
    lane_w = lane * N_WORDS
    ones = jnp.full((16,), 1, dtype=jnp.int32)

    for chunk in range(N_CHUNKS):
      row0 = rp * ROWS_PER + chunk * CHUNK
      pltpu.sync_copy(
          xw_hbm.at[pl.ds(row0 * N_WORDS, CHUNK * N_WORDS)], xw_v)

      def tree_body(tl, _):
        tref = tbl_v.at[tl]
        tlv = jnp.full((16,), tl, dtype=jnp.int32)

        @plsc.parallel_loop(0, CHUNK // 16, unroll=4)
        def rg_body(rg):
          rowb = rg * (16 * N_WORDS) + lane_w
          m = ones
          acc = jnp.zeros((16,), jnp.int32)
          for d in range(MAX_DEPTH):
            tv = plsc.load_gather(tref, [m])
            f = tv & 511
            w = plsc.load_gather(xw_v, [rowb + (f >> 5)])
            dec = (w >> (f & 31)) & 1
            acc = acc | (((tv >> dec) & 0x10000) << d)
            m = m + m + dec
          rowloc = rg * 16 + lane
          plsc.store_scatter(out_v, [rowloc, tlv], acc >> 16)

        return 0

      lax.fori_loop(0, TREES_PER, tree_body, 0)
      pltpu.sync_copy(
          out_v,
          out_hbm.at[pl.ds(row0, CHUNK), pl.ds(tp * TREES_PER, TREES_PER)])

  return forest


_forest = _make_forest_kernel()


def kernel(x, node_outputs, node_foci):
  xi = x.astype(jnp.int32).reshape(BATCH, N_WORDS, 32)
  shifts = jnp.left_shift(jnp.int32(1), jnp.arange(32, dtype=jnp.int32))
  xw = jnp.sum(xi * shifts, axis=-1, dtype=jnp.int32).reshape(-1)

  tbl = (node_foci.astype(jnp.int32)
         | (node_outputs[..., 0].astype(jnp.int32) << 16)
         | (node_outputs[..., 1].astype(jnp.int32) << 17))
  tbl = jnp.pad(tbl, ((0, 0), (1, 0)))[:, :NPAD]

  out_words = _forest(xw, tbl)

  depths = jnp.arange(MAX_DEPTH, dtype=jnp.int32)[None, :, None]
  bits = (out_words[:, None, :] >> depths) & 1
  out_bools = bits.reshape(BATCH, MAX_DEPTH * N_TREES) > 0
  x_cat = jnp.concatenate([x, out_bools], axis=1)
  output = out_bools[:, (MAX_DEPTH - 1) * N_TREES:]
  return (x_cat, output)


# speedup vs baseline: 1.0110x; 1.0110x over previous
import functools

import numpy as np
import jax
import jax.numpy as jnp
from jax import lax
from jax.experimental import pallas as pl
from jax.experimental.pallas import tpu as pltpu
from jax.experimental.pallas import tpu_sc as plsc

N_TREES = 128
MAX_DEPTH = 10
N_NODES = 2 ** MAX_DEPTH - 1
NPAD = N_NODES + 1
BATCH = 16384
N_FEAT = 512
N_WORDS = N_FEAT // 32

NC = 2
NS = 16
NW = NC * NS
TP = 4
RP = NW // TP
TREES_PER = N_TREES // TP
ROWS_PER = BATCH // RP
CHUNK = 1024
N_CHUNKS = ROWS_PER // CHUNK
UNROLL = 4


def _make_forest_kernel():
  mesh = plsc.VectorSubcoreMesh(
      core_axis_name="c", subcore_axis_name="s", num_cores=NC,
      num_subcores=NS)

  @functools.partial(
      pl.kernel,
      out_type=jax.ShapeDtypeStruct((BATCH, N_TREES), jnp.int32),
      mesh=mesh,
      scratch_types=[
          pltpu.VMEM((TREES_PER, NPAD), jnp.int32),
          pltpu.VMEM((TREES_PER, NPAD), jnp.int32),
          pltpu.VMEM((CHUNK * N_WORDS,), jnp.int32),
          pltpu.VMEM((CHUNK, TREES_PER), jnp.int32),
      ],
      compiler_params=pltpu.CompilerParams(
          use_tc_tiling_on_sc=False, needs_layout_passes=False),
  )
  def forest(xw_hbm, ftbl_hbm, lout_hbm, out_hbm, ftbl_v, lout_v, xw_v, out_v):
    wid = lax.axis_index("s") * NC + lax.axis_index("c")
    tp = wid // RP
    rp = wid % RP

    pltpu.sync_copy(ftbl_hbm.at[pl.ds(tp * TREES_PER, TREES_PER), :], ftbl_v)
    pltpu.sync_copy(lout_hbm.at[pl.ds(tp * TREES_PER, TREES_PER), :], lout_v)

    lane = lax.iota(jnp.int32, 16)
    lane_w = lane * N_WORDS
    ones = jnp.full((16,), 1, dtype=jnp.int32)

    for chunk in range(N_CHUNKS):
      row0 = rp * ROWS_PER + chunk * CHUNK
      pltpu.sync_copy(
          xw_hbm.at[pl.ds(row0 * N_WORDS, CHUNK * N_WORDS)], xw_v)

      def tree_body(tl, _):
        tref = ftbl_v.at[tl]
        lref = lout_v.at[tl]
        tlv = jnp.full((16,), tl, dtype=jnp.int32)

        @plsc.parallel_loop(0, CHUNK // 16, unroll=UNROLL)
        def rg_body(rg):
          rowb = rg * (16 * N_WORDS) + lane_w
          m = ones
          for d in range(MAX_DEPTH):
            tv = plsc.load_gather(tref, [m])
            w = plsc.load_gather(xw_v, [rowb + (tv >> 5)])
            dec = (w >> (tv & 31)) & 1
            m = m + m + dec
          acc = plsc.load_gather(lref, [m & 1023])
          rowloc = rg * 16 + lane
          plsc.store_scatter(out_v, [rowloc, tlv], acc)

        return 0

      lax.fori_loop(0, TREES_PER, tree_body, 0)
      pltpu.sync_copy(
          out_v,
          out_hbm.at[pl.ds(row0, CHUNK), pl.ds(tp * TREES_PER, TREES_PER)])

  return forest


_forest = _make_forest_kernel()


_LEAF = np.arange(1024)
_LEAF_NODE = []
_LEAF_DEC = []
_m = np.ones(1024, np.int64)
for _d in range(MAX_DEPTH):
  _b = (_LEAF >> (MAX_DEPTH - 1 - _d)) & 1
  _LEAF_NODE.append(_m - 1)
  _LEAF_DEC.append(_b)
  _m = 2 * _m + _b


def kernel(x, node_outputs, node_foci):
  xi = x.astype(jnp.int32).reshape(BATCH, N_WORDS, 32)
  shifts = jnp.left_shift(jnp.int32(1), jnp.arange(32, dtype=jnp.int32))
  xw = jnp.sum(xi * shifts, axis=-1, dtype=jnp.int32).reshape(-1)

  ftbl = jnp.pad(node_foci.astype(jnp.int32), ((0, 0), (1, 0)))[:, :NPAD]

  lout = jnp.zeros((N_TREES, NPAD), jnp.int32)
  for d in range(MAX_DEPTH):
    lout = lout | (node_outputs[:, _LEAF_NODE[d], _LEAF_DEC[d]]
                   .astype(jnp.int32) << d)

  out_words = _forest(xw, ftbl, lout)

  depths = jnp.arange(MAX_DEPTH, dtype=jnp.int32)[None, :, None]
  bits = (out_words[:, None, :] >> depths) & 1
  out_bools = bits.reshape(BATCH, MAX_DEPTH * N_TREES) > 0
  x_cat = jnp.concatenate([x, out_bools], axis=1)
  output = out_bools[:, (MAX_DEPTH - 1) * N_TREES:]
  return (x_cat, output)


# speedup vs baseline: 1.2088x; 1.1956x over previous
import functools

import numpy as np
import jax
import jax.numpy as jnp
from jax import lax
from jax.experimental import pallas as pl
from jax.experimental.pallas import tpu as pltpu
from jax.experimental.pallas import tpu_sc as plsc

N_TREES = 128
MAX_DEPTH = 10
N_NODES = 2 ** MAX_DEPTH - 1
NPAD = N_NODES + 1
BATCH = 16384
N_FEAT = 512
N_WORDS = N_FEAT // 32

NC = 2
NS = 16
NW = NC * NS
TP = 4
RP = NW // TP
TREES_PER = N_TREES // TP
ROWS_PER = BATCH // RP
CHUNK = 1024
N_CHUNKS = ROWS_PER // CHUNK
UNROLL = 4


def _make_forest_kernel():
  mesh = plsc.VectorSubcoreMesh(
      core_axis_name="c", subcore_axis_name="s", num_cores=NC,
      num_subcores=NS)

  @functools.partial(
      pl.kernel,
      out_type=jax.ShapeDtypeStruct((BATCH, N_TREES), jnp.int32),
      mesh=mesh,
      scratch_types=[
          pltpu.VMEM((TREES_PER, NPAD), jnp.int32),
          pltpu.VMEM((TREES_PER, NPAD), jnp.int32),
          pltpu.VMEM((CHUNK * N_WORDS,), jnp.int32),
          pltpu.VMEM((CHUNK, TREES_PER), jnp.int32),
      ],
      compiler_params=pltpu.CompilerParams(
          use_tc_tiling_on_sc=False, needs_layout_passes=False),
  )
  def forest(xw_hbm, ftbl_hbm, lout_hbm, out_hbm, ftbl_v, lout_v, xw_v, out_v):
    wid = lax.axis_index("s") * NC + lax.axis_index("c")
    tp = wid // RP
    rp = wid % RP

    pltpu.sync_copy(ftbl_hbm.at[pl.ds(tp * TREES_PER, TREES_PER), :], ftbl_v)
    pltpu.sync_copy(lout_hbm.at[pl.ds(tp * TREES_PER, TREES_PER), :], lout_v)

    lane = lax.iota(jnp.int32, 16)
    lane_w = lane * N_WORDS
    ones = jnp.full((16,), 1, dtype=jnp.int32)

    for chunk in range(N_CHUNKS):
      row0 = rp * ROWS_PER + chunk * CHUNK
      pltpu.sync_copy(
          xw_hbm.at[pl.ds(row0 * N_WORDS, CHUNK * N_WORDS)], xw_v)

      def tree_body(tl, _):
        tref = ftbl_v.at[tl]
        lref = lout_v.at[tl]
        tlv = jnp.full((16,), tl, dtype=jnp.int32)

        @plsc.parallel_loop(0, CHUNK // 16, unroll=UNROLL)
        def rg_body(rg):
          rowb = rg * (16 * N_WORDS) + lane_w
          m = ones
          for d in range(MAX_DEPTH):
            tv = plsc.load_gather(tref, [m])
            w = plsc.load_gather(xw_v, [rowb + (tv >> 5)])
            dec = (w >> (tv & 31)) & 1
            m = m + m + dec
          acc = plsc.load_gather(lref, [m & 1023])
          rowloc = rg * 16 + lane
          plsc.store_scatter(out_v, [rowloc, tlv], acc)

        return 0

      lax.fori_loop(0, TREES_PER, tree_body, 0)
      pltpu.sync_copy(
          out_v,
          out_hbm.at[pl.ds(row0, CHUNK), pl.ds(tp * TREES_PER, TREES_PER)])

  return forest


_forest = _make_forest_kernel()


ROWS_TC = 1024


def _assemble_body(x_ref, ow_ref, cat_ref, out_ref):
  cat_ref[:, :N_FEAT] = x_ref[...]
  ow = ow_ref[...]
  for d in range(MAX_DEPTH):
    b = ((ow >> d) & 1) != 0
    cat_ref[:, N_FEAT + d * N_TREES:N_FEAT + (d + 1) * N_TREES] = b
  out_ref[...] = b


_assemble = pl.pallas_call(
    _assemble_body,
    grid=(BATCH // ROWS_TC,),
    in_specs=[
        pl.BlockSpec((ROWS_TC, N_FEAT), lambda i: (i, 0)),
        pl.BlockSpec((ROWS_TC, N_TREES), lambda i: (i, 0)),
    ],
    out_specs=[
        pl.BlockSpec((ROWS_TC, N_FEAT + MAX_DEPTH * N_TREES),
                     lambda i: (i, 0)),
        pl.BlockSpec((ROWS_TC, N_TREES), lambda i: (i, 0)),
    ],
    out_shape=[
        jax.ShapeDtypeStruct((BATCH, N_FEAT + MAX_DEPTH * N_TREES),
                             jnp.bool_),
        jax.ShapeDtypeStruct((BATCH, N_TREES), jnp.bool_),
    ],
)


def kernel(x, node_outputs, node_foci):
  xi = x.astype(jnp.int32).reshape(BATCH, N_WORDS, 32)
  shifts = jnp.left_shift(jnp.int32(1), jnp.arange(32, dtype=jnp.int32))
  xw = jnp.sum(xi * shifts, axis=-1, dtype=jnp.int32).reshape(-1)

  ftbl = jnp.pad(node_foci.astype(jnp.int32), ((0, 0), (1, 0)))[:, :NPAD]

  lout = jnp.zeros((N_TREES, NPAD), jnp.int32)
  for d in range(MAX_DEPTH):
    slab = node_outputs[:, 2 ** d - 1:2 ** (d + 1) - 1, :].astype(jnp.int32)
    expanded = jnp.broadcast_to(
        slab.reshape(N_TREES, 2 ** (d + 1), 1),
        (N_TREES, 2 ** (d + 1), NPAD // 2 ** (d + 1)),
    ).reshape(N_TREES, NPAD)
    lout = lout | (expanded << d)

  out_words = _forest(xw, ftbl, lout)

  x_cat, output = _assemble(x, out_words)
  return (x_cat, output)
